# baseline (speedup 1.0000x reference)
import functools

import jax
import jax.numpy as jnp
from jax import lax
from jax.experimental import pallas as pl
from jax.experimental.pallas import tpu as pltpu
from jax.experimental.pallas import tpu_sc as plsc

EX = 256
N_PV = 200
EMBED_DIM = 32
FOURIER = 8
T_OUT = 12
F_OUT = 70
PACK = 4
ROWS4 = 25000

_NC = 2
_NS = 16
_NW = _NC * _NS
_CHUNKS = 20
_CHUNK = 80
_PER_W = _CHUNKS * _CHUNK

_EBLK = 8


def _sc_gather_body(table_hbm, q_hbm, out_hbm, q_v, buf0, buf1, sem0, sem1):
    wid = lax.axis_index("s") * _NC + lax.axis_index("c")
    pltpu.sync_copy(q_hbm.at[wid], q_v)
    bufs = (buf0, buf1)
    sems = (sem0, sem1)
    copies = [None, None]
    copies[0] = pltpu.async_copy(table_hbm.at[q_v.at[0]], buf0, sem0)
    for j in range(_CHUNKS):
        if j + 1 < _CHUNKS:
            copies[(j + 1) % 2] = pltpu.async_copy(
                table_hbm.at[q_v.at[j + 1]], bufs[(j + 1) % 2], sems[(j + 1) % 2])
        copies[j % 2].wait()
        pltpu.sync_copy(bufs[j % 2], out_hbm.at[wid, pl.ds(j * _CHUNK, _CHUNK)])


@jax.jit
def _sc_gather(table4, q):
    mesh = plsc.VectorSubcoreMesh(core_axis_name="c", subcore_axis_name="s")
    return pl.kernel(
        _sc_gather_body,
        out_type=jax.ShapeDtypeStruct((_NW, _PER_W, PACK * EMBED_DIM), jnp.float32),
        mesh=mesh,
        scratch_types=[
            pltpu.VMEM((_CHUNKS, _CHUNK), jnp.int32),
            pltpu.VMEM((_CHUNK, PACK * EMBED_DIM), jnp.float32),
            pltpu.VMEM((_CHUNK, PACK * EMBED_DIM), jnp.float32),
            pltpu.SemaphoreType.DMA,
            pltpu.SemaphoreType.DMA,
        ],
    )(table4, q)


def _assemble_body(pvt_ref, y_ref, x_ref, tf_ref, emb_ref, m_ref, az_ref, el_ref,
                   out_ref):
    m = m_ref[...]
    emb = emb_ref[..., 0:EMBED_DIM]
    for k in range(1, PACK):
        emb = jnp.where(m == k, emb_ref[..., k * EMBED_DIM:(k + 1) * EMBED_DIM], emb)
    tfb = jnp.broadcast_to(tf_ref[:, 0], (_EBLK, N_PV, FOURIER))
    azb = jnp.broadcast_to(az_ref[:, 0], (_EBLK, N_PV, 1))
    elb = jnp.broadcast_to(el_ref[:, 0], (_EBLK, N_PV, 1))
    tile = jnp.concatenate(
        [pvt_ref[...], y_ref[...], x_ref[...], tfb, emb, azb, elb],
        axis=-1,
    )
    out_ref[...] = jnp.where(jnp.isnan(tile), jnp.float32(0.0), tile).reshape(
        _EBLK, 1, N_PV, F_OUT)


@jax.jit
def _assemble(pvt, y, x, tf, emb_pad, m, az, el):
    grid = (EX // _EBLK, T_OUT)
    return pl.pallas_call(
        _assemble_body,
        grid=grid,
        in_specs=[
            pl.BlockSpec((_EBLK, N_PV, T_OUT), lambda i, t: (i, 0, 0)),
            pl.BlockSpec((_EBLK, N_PV, FOURIER), lambda i, t: (i, 0, 0)),
            pl.BlockSpec((_EBLK, N_PV, FOURIER), lambda i, t: (i, 0, 0)),
            pl.BlockSpec((_EBLK, 1, 1, FOURIER), lambda i, t: (i, t, 0, 0)),
            pl.BlockSpec((_EBLK, N_PV, PACK * EMBED_DIM), lambda i, t: (i, 0, 0)),
            pl.BlockSpec((_EBLK, N_PV, 1), lambda i, t: (i, 0, 0)),
            pl.BlockSpec((_EBLK, 1, 1, 1), lambda i, t: (i, t, 0, 0)),
            pl.BlockSpec((_EBLK, 1, 1, 1), lambda i, t: (i, t, 0, 0)),
        ],
        out_specs=pl.BlockSpec((_EBLK, 1, N_PV, F_OUT), lambda i, t: (i, t, 0, 0)),
        out_shape=jax.ShapeDtypeStruct((EX, T_OUT, N_PV, F_OUT), jnp.float32),
    )(pvt, y, x, tf, emb_pad, m, az, el)


def kernel(pv_y_osgb_fourier, pv_x_osgb_fourier, pv_system_row_number, pv_x_osgb, pv,
           pv_time_utc_fourier, solar_azimuth, solar_elevation, pv_system_id_embedding):
    idx = pv_system_row_number.astype(jnp.int32)
    q = (idx // PACK).reshape(_NW, _CHUNKS, _CHUNK)
    m = (idx % PACK).reshape(EX, N_PV, 1)
    table4 = pv_system_id_embedding.reshape(ROWS4, PACK * EMBED_DIM)
    emb_pad = _sc_gather(table4, q).reshape(EX, N_PV, PACK * EMBED_DIM)
    pvt = jnp.transpose(pv[:, :T_OUT], (0, 2, 1))
    tf = pv_time_utc_fourier[:, T_OUT:].reshape(EX, T_OUT, 1, FOURIER)
    az = solar_azimuth[:, T_OUT:].reshape(EX, T_OUT, 1, 1)
    el = solar_elevation[:, T_OUT:].reshape(EX, T_OUT, 1, 1)
    out = _assemble(pvt, pv_y_osgb_fourier, pv_x_osgb_fourier, tf, emb_pad, m, az, el)
    return out.reshape(EX, T_OUT * N_PV, F_OUT)


# speedup vs baseline: 1.8160x; 1.8160x over previous
import functools

import jax
import jax.numpy as jnp
from jax import lax
from jax.experimental import pallas as pl
from jax.experimental.pallas import tpu as pltpu
from jax.experimental.pallas import tpu_sc as plsc

EX = 256
N_PV = 200
EMBED_DIM = 32
FOURIER = 8
T_OUT = 12
F_OUT = 70
PACK = 4
ROWS4 = 25000

_NC = 2
_NS = 16
_NW = _NC * _NS
_CHUNKS = 20
_CHUNK = 80
_PER_W = _CHUNKS * _CHUNK

_EBLK = 8
_RBLK = 4000


def _repack_body(in_ref, out_ref):
    for k in range(PACK):
        out_ref[:, k * EMBED_DIM:(k + 1) * EMBED_DIM] = (
            in_ref[pl.ds(k, _RBLK // PACK, PACK), :])


@jax.jit
def _repack(table):
    return pl.pallas_call(
        _repack_body,
        grid=(100000 // _RBLK,),
        in_specs=[pl.BlockSpec((_RBLK, EMBED_DIM), lambda i: (i, 0))],
        out_specs=pl.BlockSpec((_RBLK // PACK, PACK * EMBED_DIM), lambda i: (i, 0)),
        out_shape=jax.ShapeDtypeStruct((ROWS4, PACK * EMBED_DIM), jnp.float32),
    )(table)


def _sc_gather_body(table_hbm, q_hbm, out_hbm, q_v, buf0, buf1, sem0, sem1):
    wid = lax.axis_index("s") * _NC + lax.axis_index("c")
    pltpu.sync_copy(q_hbm.at[wid], q_v)
    bufs = (buf0, buf1)
    sems = (sem0, sem1)
    copies = [None, None]
    copies[0] = pltpu.async_copy(table_hbm.at[q_v.at[0]], buf0, sem0)
    for j in range(_CHUNKS):
        if j + 1 < _CHUNKS:
            copies[(j + 1) % 2] = pltpu.async_copy(
                table_hbm.at[q_v.at[j + 1]], bufs[(j + 1) % 2], sems[(j + 1) % 2])
        copies[j % 2].wait()
        pltpu.sync_copy(bufs[j % 2], out_hbm.at[wid, pl.ds(j * _CHUNK, _CHUNK)])


@jax.jit
def _sc_gather(table4, q):
    mesh = plsc.VectorSubcoreMesh(core_axis_name="c", subcore_axis_name="s")
    return pl.kernel(
        _sc_gather_body,
        out_type=jax.ShapeDtypeStruct((_NW, _PER_W, PACK * EMBED_DIM), jnp.float32),
        mesh=mesh,
        scratch_types=[
            pltpu.VMEM((_CHUNKS, _CHUNK), jnp.int32),
            pltpu.VMEM((_CHUNK, PACK * EMBED_DIM), jnp.float32),
            pltpu.VMEM((_CHUNK, PACK * EMBED_DIM), jnp.float32),
            pltpu.SemaphoreType.DMA,
            pltpu.SemaphoreType.DMA,
        ],
    )(table4, q)


def _assemble_body(pvt_ref, y_ref, x_ref, tf_ref, emb_ref, m_ref, az_ref, el_ref,
                   out_ref, static_ref):
    t = pl.program_id(1)

    def clean(v):
        return jnp.where(jnp.isnan(v), jnp.float32(0.0), v)

    @pl.when(t == 0)
    def _build_static():
        m = m_ref[...]
        emb = emb_ref[..., 0:EMBED_DIM]
        for k in range(1, PACK):
            emb = jnp.where(m == k, emb_ref[..., k * EMBED_DIM:(k + 1) * EMBED_DIM],
                            emb)
        zt = jnp.zeros((_EBLK, N_PV, FOURIER), jnp.float32)
        z1 = jnp.zeros((_EBLK, N_PV, 1), jnp.float32)
        stat = jnp.concatenate(
            [pvt_ref[...], y_ref[...], x_ref[...], zt, emb, z1, z1], axis=-1)
        static_ref[...] = clean(stat)

    lane = lax.broadcasted_iota(jnp.int32, (_EBLK, 1, F_OUT), 2)
    tmask = ((lane >= 28) & (lane < 36)) | (lane >= 68)
    trow = jnp.concatenate(
        [jnp.zeros((_EBLK, 1, 28), jnp.float32), tf_ref[:, 0],
         jnp.zeros((_EBLK, 1, EMBED_DIM), jnp.float32), az_ref[:, 0], el_ref[:, 0]],
        axis=-1)
    trow = clean(trow)
    tile = jnp.where(tmask, trow, static_ref[...])
    out_ref[...] = tile.reshape(_EBLK, 1, N_PV, F_OUT)


@jax.jit
def _assemble(pvt, y, x, tf, emb_pad, m, az, el):
    grid = (EX // _EBLK, T_OUT)
    return pl.pallas_call(
        _assemble_body,
        grid=grid,
        in_specs=[
            pl.BlockSpec((_EBLK, N_PV, T_OUT), lambda i, t: (i, 0, 0)),
            pl.BlockSpec((_EBLK, N_PV, FOURIER), lambda i, t: (i, 0, 0)),
            pl.BlockSpec((_EBLK, N_PV, FOURIER), lambda i, t: (i, 0, 0)),
            pl.BlockSpec((_EBLK, 1, 1, FOURIER), lambda i, t: (i, t, 0, 0)),
            pl.BlockSpec((_EBLK, N_PV, PACK * EMBED_DIM), lambda i, t: (i, 0, 0)),
            pl.BlockSpec((_EBLK, N_PV, EMBED_DIM), lambda i, t: (i, 0, 0)),
            pl.BlockSpec((_EBLK, 1, 1, 1), lambda i, t: (i, t, 0, 0)),
            pl.BlockSpec((_EBLK, 1, 1, 1), lambda i, t: (i, t, 0, 0)),
        ],
        out_specs=pl.BlockSpec((_EBLK, 1, N_PV, F_OUT), lambda i, t: (i, t, 0, 0)),
        out_shape=jax.ShapeDtypeStruct((EX, T_OUT, N_PV, F_OUT), jnp.float32),
        scratch_shapes=[pltpu.VMEM((_EBLK, N_PV, F_OUT), jnp.float32)],
    )(pvt, y, x, tf, emb_pad, m, az, el)


def kernel(pv_y_osgb_fourier, pv_x_osgb_fourier, pv_system_row_number, pv_x_osgb, pv,
           pv_time_utc_fourier, solar_azimuth, solar_elevation, pv_system_id_embedding):
    idx = pv_system_row_number.astype(jnp.int32)
    q = (idx // PACK).reshape(_NW, _CHUNKS, _CHUNK)
    m = jnp.broadcast_to((idx % PACK)[:, :, None], (EX, N_PV, EMBED_DIM))
    table4 = _repack(pv_system_id_embedding)
    emb_pad = _sc_gather(table4, q).reshape(EX, N_PV, PACK * EMBED_DIM)
    pvt = jnp.transpose(pv[:, :T_OUT], (0, 2, 1))
    tf = pv_time_utc_fourier[:, T_OUT:].reshape(EX, T_OUT, 1, FOURIER)
    az = solar_azimuth[:, T_OUT:].reshape(EX, T_OUT, 1, 1)
    el = solar_elevation[:, T_OUT:].reshape(EX, T_OUT, 1, 1)
    out = _assemble(pvt, pv_y_osgb_fourier, pv_x_osgb_fourier, tf, emb_pad, m, az, el)
    return out.reshape(EX, T_OUT * N_PV, F_OUT)


# speedup vs baseline: 6.0087x; 3.3087x over previous
import functools

import jax
import jax.numpy as jnp
from jax import lax
from jax.experimental import pallas as pl
from jax.experimental.pallas import tpu as pltpu
from jax.experimental.pallas import tpu_sc as plsc

EX = 256
N_PV = 200
EMBED_DIM = 32
FOURIER = 8
T_OUT = 12
F_OUT = 70
REP = 4
N_TABLE = 100000

_NC = 2
_NS = 16
_NW = _NC * _NS
_CHUNKS = 20
_CHUNK = 80
_PER_W = _CHUNKS * _CHUNK

_RBLK = 5000
_NCHUNK = 40


def _repack_body(in_ref, out_ref):
    x = in_ref[...]
    for k in range(REP):
        out_ref[:, k * EMBED_DIM:(k + 1) * EMBED_DIM] = x


@jax.jit
def _repack(table):
    return pl.pallas_call(
        _repack_body,
        grid=(N_TABLE // _RBLK,),
        in_specs=[pl.BlockSpec((_RBLK, EMBED_DIM), lambda i: (i, 0))],
        out_specs=pl.BlockSpec((_RBLK, REP * EMBED_DIM), lambda i: (i, 0)),
        out_shape=jax.ShapeDtypeStruct((N_TABLE, REP * EMBED_DIM), jnp.float32),
    )(table)


def _sc_gather_body(table_hbm, idx_hbm, out_hbm, idx_v, buf0, buf1, sem0, sem1):
    wid = lax.axis_index("s") * _NC + lax.axis_index("c")
    pltpu.sync_copy(idx_hbm.at[wid], idx_v)
    bufs = (buf0, buf1)
    sems = (sem0, sem1)
    copies = [None, None]
    copies[0] = pltpu.async_copy(table_hbm.at[idx_v.at[0]], buf0, sem0)
    for j in range(_CHUNKS):
        if j + 1 < _CHUNKS:
            copies[(j + 1) % 2] = pltpu.async_copy(
                table_hbm.at[idx_v.at[j + 1]], bufs[(j + 1) % 2], sems[(j + 1) % 2])
        copies[j % 2].wait()
        pltpu.sync_copy(bufs[j % 2], out_hbm.at[wid, pl.ds(j * _CHUNK, _CHUNK)])


@jax.jit
def _sc_gather(table_rep, idx):
    mesh = plsc.VectorSubcoreMesh(core_axis_name="c", subcore_axis_name="s")
    return pl.kernel(
        _sc_gather_body,
        out_type=jax.ShapeDtypeStruct((_NW, _PER_W, REP * EMBED_DIM), jnp.float32),
        mesh=mesh,
        scratch_types=[
            pltpu.VMEM((_CHUNKS, _CHUNK), jnp.int32),
            pltpu.VMEM((_CHUNK, REP * EMBED_DIM), jnp.float32),
            pltpu.VMEM((_CHUNK, REP * EMBED_DIM), jnp.float32),
            pltpu.SemaphoreType.DMA,
            pltpu.SemaphoreType.DMA,
        ],
    )(table_rep, idx)


def _assemble_body(pvt_ref, y_ref, x_ref, emb_ref, tf_ref, az_ref, el_ref, out_ref):
    def clean(v):
        return jnp.where(jnp.isnan(v), jnp.float32(0.0), v)

    out_ref[0:12, 0] = clean(pvt_ref[...])
    out_ref[12:20, 0] = clean(y_ref[...])
    out_ref[20:28, 0] = clean(x_ref[...])
    tf = clean(tf_ref[:, 0])
    out_ref[28:36, 0] = jnp.broadcast_to(tf, (FOURIER, _NCHUNK, EX))
    out_ref[36:68, 0] = clean(emb_ref[...])
    az = clean(az_ref[...])
    el = clean(el_ref[...])
    out_ref[68:69, 0] = jnp.broadcast_to(az, (1, _NCHUNK, EX))
    out_ref[69:70, 0] = jnp.broadcast_to(el, (1, _NCHUNK, EX))


@jax.jit
def _assemble(pvt, y, x, emb, tf, az, el):
    grid = (N_PV // _NCHUNK, T_OUT)
    return pl.pallas_call(
        _assemble_body,
        grid=grid,
        in_specs=[
            pl.BlockSpec((T_OUT, _NCHUNK, EX), lambda n, t: (0, n, 0)),
            pl.BlockSpec((FOURIER, _NCHUNK, EX), lambda n, t: (0, n, 0)),
            pl.BlockSpec((FOURIER, _NCHUNK, EX), lambda n, t: (0, n, 0)),
            pl.BlockSpec((EMBED_DIM, _NCHUNK, EX), lambda n, t: (0, n, 0)),
            pl.BlockSpec((FOURIER, 1, 1, EX), lambda n, t: (0, t, 0, 0)),
            pl.BlockSpec((1, 1, EX), lambda n, t: (t, 0, 0)),
            pl.BlockSpec((1, 1, EX), lambda n, t: (t, 0, 0)),
        ],
        out_specs=pl.BlockSpec((F_OUT, 1, _NCHUNK, EX), lambda n, t: (0, t, n, 0)),
        out_shape=jax.ShapeDtypeStruct((F_OUT, T_OUT, N_PV, EX), jnp.float32),
    )(pvt, y, x, emb, tf, az, el)


def kernel(pv_y_osgb_fourier, pv_x_osgb_fourier, pv_system_row_number, pv_x_osgb, pv,
           pv_time_utc_fourier, solar_azimuth, solar_elevation, pv_system_id_embedding):
    idx = pv_system_row_number.astype(jnp.int32).reshape(_NW, _CHUNKS, _CHUNK)
    table_rep = _repack(pv_system_id_embedding)
    emb_pad = _sc_gather(table_rep, idx).reshape(EX, N_PV, REP * EMBED_DIM)
    embT = jnp.transpose(emb_pad[:, :, :EMBED_DIM], (2, 1, 0))
    pvtT = jnp.transpose(pv[:, :T_OUT], (1, 2, 0))
    yT = jnp.transpose(pv_y_osgb_fourier, (2, 1, 0))
    xT = jnp.transpose(pv_x_osgb_fourier, (2, 1, 0))
    tfT = jnp.transpose(pv_time_utc_fourier[:, T_OUT:], (2, 1, 0)).reshape(
        FOURIER, T_OUT, 1, EX)
    azT = jnp.transpose(solar_azimuth[:, T_OUT:], (1, 0)).reshape(T_OUT, 1, EX)
    elT = jnp.transpose(solar_elevation[:, T_OUT:], (1, 0)).reshape(T_OUT, 1, EX)
    outT = _assemble(pvtT, yT, xT, embT, tfT, azT, elT)
    return jnp.transpose(outT.reshape(F_OUT, T_OUT * N_PV, EX), (2, 1, 0))


# speedup vs baseline: 6.8159x; 1.1343x over previous
import functools

import jax
import jax.numpy as jnp
from jax import lax
from jax.experimental import pallas as pl
from jax.experimental.pallas import tpu as pltpu
from jax.experimental.pallas import tpu_sc as plsc

EX = 256
N_PV = 200
EMBED_DIM = 32
FOURIER = 8
T_OUT = 12
F_OUT = 70
REP = 4
N_TABLE = 100000

_NC = 2
_NS = 16
_NW = _NC * _NS
_CHUNKS = 20
_CHUNK = 80
_PER_W = _CHUNKS * _CHUNK

_RBLK = 5000
_NCHUNK = 200


def _repack_body(in_ref, out_ref):
    rep = (lax.broadcasted_iota(jnp.int32, (EMBED_DIM, REP * EMBED_DIM), 1)
           % EMBED_DIM
           == lax.broadcasted_iota(jnp.int32, (EMBED_DIM, REP * EMBED_DIM), 0)
           ).astype(jnp.float32)
    out_ref[...] = jax.lax.dot_general(
        in_ref[...], rep, (((1,), (0,)), ((), ())),
        preferred_element_type=jnp.float32)


@jax.jit
def _repack(table):
    return pl.pallas_call(
        _repack_body,
        grid=(N_TABLE // _RBLK,),
        in_specs=[pl.BlockSpec((_RBLK, EMBED_DIM), lambda i: (i, 0))],
        out_specs=pl.BlockSpec((_RBLK, REP * EMBED_DIM), lambda i: (i, 0)),
        out_shape=jax.ShapeDtypeStruct((N_TABLE, REP * EMBED_DIM), jnp.float32),
    )(table)


def _sc_gather_body(table_hbm, idx_hbm, out_hbm, idx_v, buf0, buf1, sem0, sem1):
    wid = lax.axis_index("s") * _NC + lax.axis_index("c")
    pltpu.sync_copy(idx_hbm.at[wid], idx_v)
    bufs = (buf0, buf1)
    sems = (sem0, sem1)
    copies = [None, None]
    copies[0] = pltpu.async_copy(table_hbm.at[idx_v.at[0]], buf0, sem0)
    for j in range(_CHUNKS):
        if j + 1 < _CHUNKS:
            copies[(j + 1) % 2] = pltpu.async_copy(
                table_hbm.at[idx_v.at[j + 1]], bufs[(j + 1) % 2], sems[(j + 1) % 2])
        copies[j % 2].wait()
        pltpu.sync_copy(bufs[j % 2], out_hbm.at[wid, pl.ds(j * _CHUNK, _CHUNK)])


@jax.jit
def _sc_gather(table_rep, idx):
    mesh = plsc.VectorSubcoreMesh(core_axis_name="c", subcore_axis_name="s")
    return pl.kernel(
        _sc_gather_body,
        out_type=jax.ShapeDtypeStruct((_NW, _PER_W, REP * EMBED_DIM), jnp.float32),
        mesh=mesh,
        scratch_types=[
            pltpu.VMEM((_CHUNKS, _CHUNK), jnp.int32),
            pltpu.VMEM((_CHUNK, REP * EMBED_DIM), jnp.float32),
            pltpu.VMEM((_CHUNK, REP * EMBED_DIM), jnp.float32),
            pltpu.SemaphoreType.DMA,
            pltpu.SemaphoreType.DMA,
        ],
    )(table_rep, idx)


def _assemble_body(pvt_ref, y_ref, x_ref, emb_ref, tf_ref, az_ref, el_ref, out_ref):
    def clean(v):
        return jnp.where(jnp.isnan(v), jnp.float32(0.0), v)

    out_ref[0:12, 0] = clean(pvt_ref[...])
    out_ref[12:20, 0] = clean(y_ref[...])
    out_ref[20:28, 0] = clean(x_ref[...])
    tf = clean(tf_ref[:, 0])
    out_ref[28:36, 0] = jnp.broadcast_to(tf, (FOURIER, _NCHUNK, EX))
    out_ref[36:68, 0] = clean(emb_ref[...])
    az = clean(az_ref[...])
    el = clean(el_ref[...])
    out_ref[68:69, 0] = jnp.broadcast_to(az, (1, _NCHUNK, EX))
    out_ref[69:70, 0] = jnp.broadcast_to(el, (1, _NCHUNK, EX))


@jax.jit
def _assemble(pvt, y, x, emb, tf, az, el):
    grid = (N_PV // _NCHUNK, T_OUT)
    return pl.pallas_call(
        _assemble_body,
        grid=grid,
        in_specs=[
            pl.BlockSpec((T_OUT, _NCHUNK, EX), lambda n, t: (0, n, 0)),
            pl.BlockSpec((FOURIER, _NCHUNK, EX), lambda n, t: (0, n, 0)),
            pl.BlockSpec((FOURIER, _NCHUNK, EX), lambda n, t: (0, n, 0)),
            pl.BlockSpec((EMBED_DIM, _NCHUNK, EX), lambda n, t: (0, n, 0)),
            pl.BlockSpec((FOURIER, 1, 1, EX), lambda n, t: (0, t, 0, 0)),
            pl.BlockSpec((1, 1, EX), lambda n, t: (t, 0, 0)),
            pl.BlockSpec((1, 1, EX), lambda n, t: (t, 0, 0)),
        ],
        out_specs=pl.BlockSpec((F_OUT, 1, _NCHUNK, EX), lambda n, t: (0, t, n, 0)),
        out_shape=jax.ShapeDtypeStruct((F_OUT, T_OUT, N_PV, EX), jnp.float32),
    )(pvt, y, x, emb, tf, az, el)


def kernel(pv_y_osgb_fourier, pv_x_osgb_fourier, pv_system_row_number, pv_x_osgb, pv,
           pv_time_utc_fourier, solar_azimuth, solar_elevation, pv_system_id_embedding):
    idx = pv_system_row_number.astype(jnp.int32).reshape(_NW, _CHUNKS, _CHUNK)
    table_rep = _repack(pv_system_id_embedding)
    emb_pad = _sc_gather(table_rep, idx).reshape(EX, N_PV, REP * EMBED_DIM)
    embT = jnp.transpose(emb_pad[:, :, :EMBED_DIM], (2, 1, 0))
    pvtT = jnp.transpose(pv[:, :T_OUT], (1, 2, 0))
    yT = jnp.transpose(pv_y_osgb_fourier, (2, 1, 0))
    xT = jnp.transpose(pv_x_osgb_fourier, (2, 1, 0))
    tfT = jnp.transpose(pv_time_utc_fourier[:, T_OUT:], (2, 1, 0)).reshape(
        FOURIER, T_OUT, 1, EX)
    azT = jnp.transpose(solar_azimuth[:, T_OUT:], (1, 0)).reshape(T_OUT, 1, EX)
    elT = jnp.transpose(solar_elevation[:, T_OUT:], (1, 0)).reshape(T_OUT, 1, EX)
    outT = _assemble(pvtT, yT, xT, embT, tfT, azT, elT)
    return jnp.transpose(outT.reshape(F_OUT, T_OUT * N_PV, EX), (2, 1, 0))
